# baseline (speedup 1.0000x reference)
import jax
import jax.numpy as jnp
from jax import lax
from jax.experimental import pallas as pl
from jax.experimental.pallas import tpu as pltpu
from jax.experimental.pallas import tpu_sc as plsc

N_GROUP = 8
TOPK_GROUP = 3
TOP_K = 8
EXP_PER_STEP = 2
LANES = 16


def _make_logits_body(nchunk):
    def _logits_body(x_ref, gw_ref, out_ref):
        lt = jax.lax.dot_general(
            gw_ref[...], x_ref[...], (((1,), (1,)), ((), ())),
            preferred_element_type=jnp.float32,
        )
        for c in range(nchunk):
            out_ref[c] = lt[:, c * LANES:(c + 1) * LANES]

    return _logits_body


def _shared_body(x_ref, sgu_ref, sdn_ref, so_ref):
    xb = x_ref[...].astype(jnp.bfloat16)
    ff2 = sgu_ref.shape[0] // 2
    sh = jax.lax.dot_general(
        xb, sgu_ref[...].astype(jnp.bfloat16),
        (((1,), (1,)), ((), ())), preferred_element_type=jnp.float32,
    )
    act = (jax.nn.silu(sh[:, :ff2]) * sh[:, ff2:]).astype(jnp.bfloat16)
    so_ref[...] = jax.lax.dot_general(
        act, sdn_ref[...].astype(jnp.bfloat16),
        (((1,), (1,)), ((), ())), preferred_element_type=jnp.float32,
    )


def _make_sc_router(m, n_e):
    gsz = n_e // N_GROUP
    nchunk = m // LANES

    def _sc_router_body(logits_hbm, comb_hbm, lg_v, cb_v):
        wid = lax.axis_index("s") * 2 + lax.axis_index("c")

        @pl.when(wid < nchunk)
        def _():
            pltpu.sync_copy(logits_hbm.at[wid], lg_v)

            zero = jnp.full((LANES,), 0.0, jnp.float32)
            one = jnp.full((LANES,), 1.0, jnp.float32)
            neg1 = jnp.full((LANES,), -1.0, jnp.float32)
            half = jnp.full((LANES,), -0.5, jnp.float32)
            kgrp = jnp.full((LANES,), float(TOPK_GROUP), jnp.float32)

            s = []
            rowmax = None
            for e in range(n_e):
                v = lg_v[e]
                s.append(v)
                rowmax = v if rowmax is None else jnp.maximum(rowmax, v)
            ex = [jnp.exp(v - rowmax) for v in s]
            tot = ex[0]
            for e in range(1, n_e):
                tot = tot + ex[e]
            sc = [v / tot for v in ex]

            gm = []
            for g in range(N_GROUP):
                gmax = sc[g * gsz]
                for jj in range(1, gsz):
                    gmax = jnp.maximum(gmax, sc[g * gsz + jj])
                gm.append(gmax)
            selg = []
            for g in range(N_GROUP):
                cnt = zero
                for g2 in range(N_GROUP):
                    cnt = cnt + jnp.where(gm[g2] > gm[g], one, zero)
                selg.append(cnt < kgrp)

            tmp = [jnp.where(selg[e // gsz], sc[e], zero) for e in range(n_e)]
            work = list(tmp)
            for _ in range(TOP_K):
                cur = work[0]
                for e in range(1, n_e):
                    cur = jnp.maximum(cur, work[e])
                for e in range(n_e):
                    work[e] = jnp.where(work[e] == cur, neg1, work[e])

            for e in range(n_e):
                cb_v[e] = jnp.where(work[e] < half, tmp[e], zero)
            pltpu.sync_copy(cb_v, comb_hbm.at[wid])

    mesh = plsc.VectorSubcoreMesh(core_axis_name="c", subcore_axis_name="s")
    return pl.kernel(
        _sc_router_body,
        out_type=jax.ShapeDtypeStruct((nchunk, n_e, LANES), jnp.float32),
        mesh=mesh,
        scratch_types=[
            pltpu.VMEM((n_e, LANES), jnp.float32),
            pltpu.VMEM((n_e, LANES), jnp.float32),
        ],
    )


def _expert_body(x_ref, comb_ref, shared_ref, g_ref, u_ref, dn_ref, out_ref):
    i = pl.program_id(0)
    xb = x_ref[...].astype(jnp.bfloat16)
    comb = comb_ref[...]
    lane = jax.lax.broadcasted_iota(jnp.int32, comb.shape, 1)

    contrib = None
    for j in range(EXP_PER_STEP):
        e = i * EXP_PER_STEP + j
        hg = jax.lax.dot_general(
            xb, g_ref[j].astype(jnp.bfloat16), (((1,), (1,)), ((), ())),
            preferred_element_type=jnp.float32,
        )
        hu = jax.lax.dot_general(
            xb, u_ref[j].astype(jnp.bfloat16), (((1,), (1,)), ((), ())),
            preferred_element_type=jnp.float32,
        )
        act = jax.nn.silu(hg) * hu

        col = jnp.sum(jnp.where(lane == e, comb, 0.0), axis=1, keepdims=True)
        actw = (act * col).astype(jnp.bfloat16)
        c = jax.lax.dot_general(
            actw, dn_ref[j].astype(jnp.bfloat16),
            (((1,), (1,)), ((), ())), preferred_element_type=jnp.float32,
        )
        contrib = c if contrib is None else contrib + c

    @pl.when(i == 0)
    def _():
        out_ref[...] = shared_ref[...] + contrib

    @pl.when(i > 0)
    def _():
        out_ref[...] = out_ref[...] + contrib


def kernel(x, gate_w, gate_up_proj, down_proj, shared_gate_up, shared_down):
    m, hidden = x.shape
    n_e, two_ff, _ = gate_up_proj.shape
    ff = down_proj.shape[2]
    nchunk = m // LANES

    logits_c = pl.pallas_call(
        _make_logits_body(nchunk),
        out_shape=jax.ShapeDtypeStruct((nchunk, n_e, LANES), jnp.float32),
    )(x, gate_w)

    comb_c = _make_sc_router(m, n_e)(logits_c)
    combine = comb_c.transpose(0, 2, 1).reshape(m, n_e)

    shared_out = pl.pallas_call(
        _shared_body,
        out_shape=jax.ShapeDtypeStruct((m, hidden), jnp.float32),
    )(x, shared_gate_up, shared_down)

    out = pl.pallas_call(
        _expert_body,
        grid=(n_e // EXP_PER_STEP,),
        in_specs=[
            pl.BlockSpec((m, hidden), lambda e: (0, 0)),
            pl.BlockSpec((m, n_e), lambda e: (0, 0)),
            pl.BlockSpec((m, hidden), lambda e: (0, 0)),
            pl.BlockSpec((EXP_PER_STEP, two_ff // 2, hidden), lambda e: (e, 0, 0)),
            pl.BlockSpec((EXP_PER_STEP, two_ff // 2, hidden), lambda e: (e, 1, 0)),
            pl.BlockSpec((EXP_PER_STEP, hidden, ff), lambda e: (e, 0, 0)),
        ],
        out_specs=pl.BlockSpec((m, hidden), lambda e: (0, 0)),
        out_shape=jax.ShapeDtypeStruct((m, hidden), jnp.float32),
    )(x, combine, shared_out, gate_up_proj, gate_up_proj, down_proj)
    return out


# speedup vs baseline: 1.0073x; 1.0073x over previous
import jax
import jax.numpy as jnp
from jax import lax
from jax.experimental import pallas as pl
from jax.experimental.pallas import tpu as pltpu
from jax.experimental.pallas import tpu_sc as plsc

N_GROUP = 8
TOPK_GROUP = 3
TOP_K = 8
EXP_PER_STEP = 2
LANES = 16


def _make_logits_body(nchunk):
    def _logits_body(x_ref, gw_ref, out_ref):
        lt = jax.lax.dot_general(
            gw_ref[...], x_ref[...], (((1,), (1,)), ((), ())),
            preferred_element_type=jnp.float32,
        )
        for c in range(nchunk):
            out_ref[c] = lt[:, c * LANES:(c + 1) * LANES]

    return _logits_body


def _shared_body(x_ref, sgu_ref, sdn_ref, so_ref):
    xb = x_ref[...].astype(jnp.bfloat16)
    ff2 = sgu_ref.shape[0] // 2
    sh = jax.lax.dot_general(
        xb, sgu_ref[...].astype(jnp.bfloat16),
        (((1,), (1,)), ((), ())), preferred_element_type=jnp.float32,
    )
    act = (jax.nn.silu(sh[:, :ff2]) * sh[:, ff2:]).astype(jnp.bfloat16)
    so_ref[...] = jax.lax.dot_general(
        act, sdn_ref[...].astype(jnp.bfloat16),
        (((1,), (1,)), ((), ())), preferred_element_type=jnp.float32,
    )


def _make_sc_router(m, n_e):
    gsz = n_e // N_GROUP
    nchunk = m // LANES

    def _sc_router_body(logits_hbm, comb_hbm, lg_v, cb_v):
        wid = lax.axis_index("s") * 2 + lax.axis_index("c")

        @pl.when(wid < nchunk)
        def _():
            pltpu.sync_copy(logits_hbm.at[wid], lg_v)

            zero = jnp.full((LANES,), 0.0, jnp.float32)
            one = jnp.full((LANES,), 1.0, jnp.float32)
            neg1 = jnp.full((LANES,), -1.0, jnp.float32)
            half = jnp.full((LANES,), -0.5, jnp.float32)
            kgrp = jnp.full((LANES,), float(TOPK_GROUP), jnp.float32)

            s = []
            rowmax = None
            for e in range(n_e):
                v = lg_v[e]
                s.append(v)
                rowmax = v if rowmax is None else jnp.maximum(rowmax, v)
            ex = [jnp.exp(v - rowmax) for v in s]
            tot = ex[0]
            for e in range(1, n_e):
                tot = tot + ex[e]
            sc = [v / tot for v in ex]

            gm = []
            for g in range(N_GROUP):
                gmax = sc[g * gsz]
                for jj in range(1, gsz):
                    gmax = jnp.maximum(gmax, sc[g * gsz + jj])
                gm.append(gmax)
            selg = []
            for g in range(N_GROUP):
                cnt = zero
                for g2 in range(N_GROUP):
                    cnt = cnt + jnp.where(gm[g2] > gm[g], one, zero)
                selg.append(cnt < kgrp)

            tmp = [jnp.where(selg[e // gsz], sc[e], zero) for e in range(n_e)]
            work = list(tmp)
            for _ in range(TOP_K):
                cur = work[0]
                for e in range(1, n_e):
                    cur = jnp.maximum(cur, work[e])
                for e in range(n_e):
                    work[e] = jnp.where(work[e] == cur, neg1, work[e])

            for e in range(n_e):
                cb_v[e] = jnp.where(work[e] < half, tmp[e], zero)
            pltpu.sync_copy(cb_v, comb_hbm.at[wid])

    mesh = plsc.VectorSubcoreMesh(core_axis_name="c", subcore_axis_name="s")
    return pl.kernel(
        _sc_router_body,
        out_type=jax.ShapeDtypeStruct((nchunk, n_e, LANES), jnp.float32),
        mesh=mesh,
        scratch_types=[
            pltpu.VMEM((n_e, LANES), jnp.float32),
            pltpu.VMEM((n_e, LANES), jnp.float32),
        ],
    )


def _make_expert_body(nchunk):
    def _expert_body(x_ref, comb3_ref, shared_ref, g_ref, u_ref, dn_ref,
                     out_ref, combt_scr):
        i = pl.program_id(0)
        n_e = combt_scr.shape[0]

        @pl.when(i == 0)
        def _():
            for c in range(nchunk):
                combt_scr[:, c * LANES:(c + 1) * LANES] = comb3_ref[c]

        xb = x_ref[...].astype(jnp.bfloat16)
        combt = combt_scr[...]
        eidx = jax.lax.broadcasted_iota(jnp.int32, (n_e, 1), 0)

        contrib = None
        for j in range(EXP_PER_STEP):
            e = i * EXP_PER_STEP + j
            hg = jax.lax.dot_general(
                xb, g_ref[j].astype(jnp.bfloat16), (((1,), (1,)), ((), ())),
                preferred_element_type=jnp.float32,
            )
            hu = jax.lax.dot_general(
                xb, u_ref[j].astype(jnp.bfloat16), (((1,), (1,)), ((), ())),
                preferred_element_type=jnp.float32,
            )
            act = jax.nn.silu(hg) * hu

            onehot = (eidx == e).astype(jnp.float32)
            col = jax.lax.dot_general(
                combt, onehot, (((0,), (0,)), ((), ())),
                preferred_element_type=jnp.float32,
            )
            actw = (act * col).astype(jnp.bfloat16)
            c = jax.lax.dot_general(
                actw, dn_ref[j].astype(jnp.bfloat16),
                (((1,), (1,)), ((), ())), preferred_element_type=jnp.float32,
            )
            contrib = c if contrib is None else contrib + c

        @pl.when(i == 0)
        def _():
            out_ref[...] = shared_ref[...] + contrib

        @pl.when(i > 0)
        def _():
            out_ref[...] = out_ref[...] + contrib

    return _expert_body


def kernel(x, gate_w, gate_up_proj, down_proj, shared_gate_up, shared_down):
    m, hidden = x.shape
    n_e, two_ff, _ = gate_up_proj.shape
    ff = down_proj.shape[2]
    nchunk = m // LANES

    logits_c = pl.pallas_call(
        _make_logits_body(nchunk),
        out_shape=jax.ShapeDtypeStruct((nchunk, n_e, LANES), jnp.float32),
    )(x, gate_w)

    comb_c = _make_sc_router(m, n_e)(logits_c)

    shared_out = pl.pallas_call(
        _shared_body,
        out_shape=jax.ShapeDtypeStruct((m, hidden), jnp.float32),
    )(x, shared_gate_up, shared_down)

    out = pl.pallas_call(
        _make_expert_body(nchunk),
        grid=(n_e // EXP_PER_STEP,),
        in_specs=[
            pl.BlockSpec((m, hidden), lambda e: (0, 0)),
            pl.BlockSpec((nchunk, n_e, LANES), lambda e: (0, 0, 0)),
            pl.BlockSpec((m, hidden), lambda e: (0, 0)),
            pl.BlockSpec((EXP_PER_STEP, two_ff // 2, hidden), lambda e: (e, 0, 0)),
            pl.BlockSpec((EXP_PER_STEP, two_ff // 2, hidden), lambda e: (e, 1, 0)),
            pl.BlockSpec((EXP_PER_STEP, hidden, ff), lambda e: (e, 0, 0)),
        ],
        out_specs=pl.BlockSpec((m, hidden), lambda e: (0, 0)),
        out_shape=jax.ShapeDtypeStruct((m, hidden), jnp.float32),
        scratch_shapes=[pltpu.VMEM((n_e, m), jnp.float32)],
    )(x, comb_c, shared_out, gate_up_proj, gate_up_proj, down_proj)
    return out
